# baseline (speedup 1.0000x reference)
import functools

import jax
import jax.numpy as jnp
from jax import lax
from jax.experimental import pallas as pl
from jax.experimental.pallas import tpu as pltpu
from jax.experimental.pallas import tpu_sc as plsc

_NC = 2
_NS = 16
_NW = _NC * _NS


def _make_sc_gather(B, W):
  assert B % _NW == 0
  bpw = B // _NW
  mesh = plsc.VectorSubcoreMesh(core_axis_name="c", subcore_axis_name="s")

  nch = 4
  ch = bpw // nch

  @functools.partial(
      pl.kernel,
      mesh=mesh,
      out_type=jax.ShapeDtypeStruct((B, W), jnp.float32),
      scratch_types=[
          pltpu.VMEM((bpw,), jnp.int32),
          pltpu.VMEM((ch, W), jnp.float32),
          pltpu.VMEM((ch, W), jnp.float32),
          pltpu.SemaphoreType.DMA,
          pltpu.SemaphoreType.DMA,
      ],
  )
  def gather1(tab_hbm, idx_hbm, out, idx_v, r0, r1, s0, s1):
    wid = lax.axis_index("s") * _NC + lax.axis_index("c")
    base = wid * bpw
    pltpu.sync_copy(idx_hbm.at[pl.ds(base, bpw)], idx_v)
    rows = (r0, r1)
    sems = (s0, s1)
    pend = [None, None]
    for c in range(nch):
      b = c % 2
      pend[b] = pltpu.async_copy(
          tab_hbm.at[idx_v.at[pl.ds(c * ch, ch)]], rows[b], sems[b])
      if c >= 1:
        pb = (c - 1) % 2
        pend[pb].wait()
        pltpu.sync_copy(rows[pb], out.at[pl.ds(base + (c - 1) * ch, ch)])
    lb = (nch - 1) % 2
    pend[lb].wait()
    pltpu.sync_copy(rows[lb], out.at[pl.ds(base + (nch - 1) * ch, ch)])

  return gather1


_WIDEN_C = 2048


def _widen_prod_body(mp_ref, ml_ref, pw_ref):
  cat = jnp.concatenate([mp_ref[...], ml_ref[...]], axis=0)
  i0 = lax.broadcasted_iota(jnp.int32, (128, 128), 0)
  i1 = lax.broadcasted_iota(jnp.int32, (128, 128), 1)
  eye = (i0 == i1).astype(jnp.float32)
  pw_ref[...] = lax.dot_general(cat, eye, (((0,), (0,)), ((), ())),
                                preferred_element_type=jnp.float32)


def _widen_user_body(u_ref, uw_ref):
  i0 = lax.broadcasted_iota(jnp.int32, (64, 128), 0)
  i1 = lax.broadcasted_iota(jnp.int32, (64, 128), 1)
  eye = (i0 == i1).astype(jnp.float32)
  uw_ref[...] = lax.dot_general(u_ref[...], eye, (((0,), (0,)), ((), ())),
                                preferred_element_type=jnp.float32)


def _widen_prod(mpT, mlT):
  D, V = mpT.shape
  C = _WIDEN_C
  return pl.pallas_call(
      _widen_prod_body,
      grid=(pl.cdiv(V, C),),
      in_specs=[pl.BlockSpec((D, C), lambda i: (0, i))] * 2,
      out_specs=pl.BlockSpec((C, 2 * D), lambda i: (i, 0)),
      out_shape=jax.ShapeDtypeStruct((V, 2 * D), jnp.float32),
  )(mpT, mlT)


def _widen_user(uT):
  D, V = uT.shape
  C = _WIDEN_C
  return pl.pallas_call(
      _widen_user_body,
      grid=(pl.cdiv(V, C),),
      in_specs=[pl.BlockSpec((D, C), lambda i: (0, i))],
      out_specs=pl.BlockSpec((C, 2 * D), lambda i: (i, 0)),
      out_shape=jax.ShapeDtypeStruct((V, 2 * D), jnp.float32),
  )(uT)


def _ln(x, g, b, eps=1e-5):
  n = x.shape[-1]
  ones = jnp.full((n, 1), 1.0 / n, jnp.float32)
  dn = (((1,), (0,)), ((), ()))
  m = lax.dot_general(x, ones, dn, preferred_element_type=jnp.float32)
  q = lax.dot_general(x * x, ones, dn, preferred_element_type=jnp.float32)
  s = lax.rsqrt(q - m * m + eps)
  sg = lax.dot_general(s, g, dn, preferred_element_type=jnp.float32)
  mg = lax.dot_general(m * s, g, dn, preferred_element_type=jnp.float32)
  return x * sg - mg + b


def _dot_t(x, w):
  return lax.dot_general(x, w, (((1,), (1,)), ((), ())),
                         preferred_element_type=jnp.float32)


def _dot_tb(x, w_bf):
  return lax.dot_general(x.astype(jnp.bfloat16), w_bf,
                         (((1,), (1,)), ((), ())),
                         preferred_element_type=jnp.float32)


def _tc_body(u_rows, p_rows,
             mf_gb, mlp_gb, wv, wo, bvo, mf_w,
             fc_w, fc_vecs, l1_w, l1_vecs, l2_w, l2_vecs,
             mlp_w, out_ref):
  mf_g = mf_gb[0:1, :]
  mf_b = mf_gb[1:2, :]
  mlp_g = mlp_gb[0:1, :]
  mlp_b = mlp_gb[1:2, :]

  u_mf = u_rows[:, :64]
  p_mf = p_rows[:, :64]
  p_mlp = p_rows[:, 64:]

  mf_vec = _ln(u_mf, mf_g, mf_b) * _ln(p_mf, mf_g, mf_b)
  mf_term = _dot_t(mf_vec, mf_w[...])

  x = _ln(p_mlp, mlp_g, mlp_b)
  a = _dot_tb(x, wv[...]) + bvo[0:1, :]
  a = _dot_tb(a, wo[...]) + bvo[1:2, :]

  h = _ln(jax.nn.relu(_dot_tb(a, fc_w[...]) + fc_vecs[0:1, :]),
          fc_vecs[1:2, :], fc_vecs[2:3, :])
  h = _ln(jax.nn.relu(_dot_tb(h, l1_w[...]) + l1_vecs[0:1, :]),
          l1_vecs[1:2, :], l1_vecs[2:3, :])
  h = _ln(jax.nn.relu(_dot_tb(h, l2_w[...]) + l2_vecs[0:1, :]),
          l2_vecs[1:2, :], l2_vecs[2:3, :])
  mlp_term = _dot_t(h, mlp_w[0:1, :])

  logit = mf_term + mlp_term + mlp_w[1, 0]
  out_ref[...] = jax.nn.sigmoid(logit)


def _full(shape):
  return pl.BlockSpec(shape, lambda i: (0, 0))


def kernel(params, user_id, product_id):
  p = params
  B = user_id.shape[0]
  D = p["mf_user"].shape[1]

  uid = user_id.astype(jnp.int32)
  pid = product_id.astype(jnp.int32)

  gather = _make_sc_gather(B, 2 * D)
  prod_w = _widen_prod(p["mf_prod"].T, p["mlp_prod"].T)
  p_rows = gather(prod_w, pid)
  user_w = _widen_user(p["mf_user"].T)
  u_rows = gather(user_w, uid)

  a = p["attn"]
  f0 = p["final_W"][0, 0]
  f1 = p["final_W"][0, 1]
  mf_w = (f0 * p["mf_out_W"][0])[None, :]
  bias_total = (f0 * p["mf_out_b"][0] + f1 * p["mlp_out_b"][0]
                + p["final_b"][0])
  mlp_w = jnp.stack([f1 * p["mlp_out_W"][0],
                     jnp.full((D,), bias_total, jnp.float32)])

  mf_gb = jnp.stack([p["mf_g"], p["mf_b"]])
  mlp_gb = jnp.stack([p["mlp_g"], p["mlp_b"]])
  bvo = jnp.stack([a["bv"], a["bo"]])
  H0, H1, H2 = p["fc_W"].shape[0], p["l1_W"].shape[0], p["l2_W"].shape[0]
  fc_w = p["fc_W"][:, :D]
  fc_vecs = jnp.stack([p["fc_b"], p["fc_g"], p["fc_beta"]])
  l1_vecs = jnp.stack([p["l1_b"], p["l1_g"], p["l1_beta"]])
  l2_vecs = jnp.stack([p["l2_b"], p["l2_g"], p["l2_beta"]])

  BLK = 4096
  grid = B // BLK
  row_spec = pl.BlockSpec((BLK, 2 * D), lambda i: (i, 0))

  out = pl.pallas_call(
      _tc_body,
      grid=(grid,),
      in_specs=[
          row_spec, row_spec,
          _full((2, D)), _full((2, D)),
          _full((D, D)), _full((D, D)), _full((2, D)), _full((1, D)),
          _full((H0, D)), _full((3, H0)),
          _full((H1, H0)), _full((3, H1)),
          _full((H2, H1)), _full((3, H2)),
          _full((2, D)),
      ],
      out_specs=pl.BlockSpec((BLK, 1), lambda i: (i, 0)),
      out_shape=jax.ShapeDtypeStruct((B, 1), jnp.float32),
  )(u_rows, p_rows,
    mf_gb, mlp_gb,
    a["Wv"].astype(jnp.bfloat16), a["Wo"].astype(jnp.bfloat16), bvo, mf_w,
    fc_w.astype(jnp.bfloat16), fc_vecs,
    p["l1_W"].astype(jnp.bfloat16), l1_vecs,
    p["l2_W"].astype(jnp.bfloat16), l2_vecs,
    mlp_w)
  return out


# speedup vs baseline: 1.0706x; 1.0706x over previous
import functools

import jax
import jax.numpy as jnp
from jax import lax
from jax.experimental import pallas as pl
from jax.experimental.pallas import tpu as pltpu
from jax.experimental.pallas import tpu_sc as plsc

_NC = 2
_NS = 16
_NW = _NC * _NS


def _make_sc_gather(B, W):
  assert B % _NW == 0
  bpw = B // _NW
  mesh = plsc.VectorSubcoreMesh(core_axis_name="c", subcore_axis_name="s")

  nch = 4
  ch = bpw // nch

  @functools.partial(
      pl.kernel,
      mesh=mesh,
      out_type=jax.ShapeDtypeStruct((B, W), jnp.float32),
      scratch_types=[
          pltpu.VMEM((bpw,), jnp.int32),
          pltpu.VMEM((ch, W), jnp.float32),
          pltpu.VMEM((ch, W), jnp.float32),
          pltpu.SemaphoreType.DMA,
          pltpu.SemaphoreType.DMA,
      ],
  )
  def gather1(tab_hbm, idx_hbm, out, idx_v, r0, r1, s0, s1):
    wid = lax.axis_index("s") * _NC + lax.axis_index("c")
    base = wid * bpw
    pltpu.sync_copy(idx_hbm.at[pl.ds(base, bpw)], idx_v)
    rows = (r0, r1)
    sems = (s0, s1)
    pend = [None, None]
    for c in range(nch):
      b = c % 2
      pend[b] = pltpu.async_copy(
          tab_hbm.at[idx_v.at[pl.ds(c * ch, ch)]], rows[b], sems[b])
      if c >= 1:
        pb = (c - 1) % 2
        pend[pb].wait()
        pltpu.sync_copy(rows[pb], out.at[pl.ds(base + (c - 1) * ch, ch)])
    lb = (nch - 1) % 2
    pend[lb].wait()
    pltpu.sync_copy(rows[lb], out.at[pl.ds(base + (nch - 1) * ch, ch)])

  return gather1


_WIDEN_C = 2048


def _widen_prod_body(mp_ref, ml_ref, pw_ref):
  cat = jnp.concatenate([mp_ref[...], ml_ref[...]], axis=0)
  i0 = lax.broadcasted_iota(jnp.int32, (128, 128), 0)
  i1 = lax.broadcasted_iota(jnp.int32, (128, 128), 1)
  eye = (i0 == i1).astype(jnp.float32)
  pw_ref[...] = lax.dot_general(cat, eye, (((0,), (0,)), ((), ())),
                                preferred_element_type=jnp.float32)


_UHALF = 51200


def _widen_prod(mpT, mlT):
  D, V = mpT.shape
  C = _WIDEN_C
  return pl.pallas_call(
      _widen_prod_body,
      grid=(pl.cdiv(V, C),),
      in_specs=[pl.BlockSpec((D, C), lambda i: (0, i))] * 2,
      out_specs=pl.BlockSpec((C, 2 * D), lambda i: (i, 0)),
      out_shape=jax.ShapeDtypeStruct((V, 2 * D), jnp.float32),
  )(mpT, mlT)


def _widen_user(uT):
  D, V = uT.shape
  C = _WIDEN_C
  hb = _UHALF // C
  lastb = (V - 1) // C

  def body(a_ref, b_ref, uw_ref):
    i = pl.program_id(0)
    base2 = jnp.minimum(hb + i, lastb) * C
    col = base2 + lax.broadcasted_iota(jnp.int32, (D, C), 1)
    b = jnp.where(col < V, b_ref[...], 0.0)
    cat = jnp.concatenate([a_ref[...], b], axis=0)
    i0 = lax.broadcasted_iota(jnp.int32, (2 * D, 2 * D), 0)
    i1 = lax.broadcasted_iota(jnp.int32, (2 * D, 2 * D), 1)
    eye = (i0 == i1).astype(jnp.float32)
    uw_ref[...] = lax.dot_general(cat, eye, (((0,), (0,)), ((), ())),
                                  preferred_element_type=jnp.float32)

  return pl.pallas_call(
      body,
      grid=(hb,),
      in_specs=[
          pl.BlockSpec((D, C), lambda i: (0, i)),
          pl.BlockSpec((D, C), lambda i: (0, jnp.minimum(hb + i, lastb))),
      ],
      out_specs=pl.BlockSpec((C, 2 * D), lambda i: (i, 0)),
      out_shape=jax.ShapeDtypeStruct((_UHALF, 2 * D), jnp.float32),
  )(uT, uT)


def _ln(x, g, b, eps=1e-5):
  n = x.shape[-1]
  ones = jnp.full((n, 1), 1.0 / n, jnp.float32)
  dn = (((1,), (0,)), ((), ()))
  m = lax.dot_general(x, ones, dn, preferred_element_type=jnp.float32)
  q = lax.dot_general(x * x, ones, dn, preferred_element_type=jnp.float32)
  s = lax.rsqrt(q - m * m + eps)
  sg = lax.dot_general(s, g, dn, preferred_element_type=jnp.float32)
  mg = lax.dot_general(m * s, g, dn, preferred_element_type=jnp.float32)
  return x * sg - mg + b


def _dot_t(x, w):
  return lax.dot_general(x, w, (((1,), (1,)), ((), ())),
                         preferred_element_type=jnp.float32)


def _dot_tb(x, w_bf):
  return lax.dot_general(x.astype(jnp.bfloat16), w_bf,
                         (((1,), (1,)), ((), ())),
                         preferred_element_type=jnp.float32)


def _tc_body(u_rows, p_rows, uflag,
             mf_gb, mlp_gb, wv, wo, bvo, mf_w,
             fc_w, fc_vecs, l1_w, l1_vecs, l2_w, l2_vecs,
             mlp_w, out_ref):
  mf_g = mf_gb[0:1, :]
  mf_b = mf_gb[1:2, :]
  mlp_g = mlp_gb[0:1, :]
  mlp_b = mlp_gb[1:2, :]

  uL = u_rows[:, :64]
  uR = u_rows[:, 64:]
  fb = lax.dot_general(uflag[...], jnp.ones((1, 64), jnp.float32),
                       (((1,), (0,)), ((), ())),
                       preferred_element_type=jnp.float32)
  u_mf = uL + fb * (uR - uL)
  p_mf = p_rows[:, :64]
  p_mlp = p_rows[:, 64:]

  mf_vec = _ln(u_mf, mf_g, mf_b) * _ln(p_mf, mf_g, mf_b)
  mf_term = _dot_t(mf_vec, mf_w[...])

  x = _ln(p_mlp, mlp_g, mlp_b)
  a = _dot_tb(x, wv[...]) + bvo[0:1, :]
  a = _dot_tb(a, wo[...]) + bvo[1:2, :]

  h = _ln(jax.nn.relu(_dot_tb(a, fc_w[...]) + fc_vecs[0:1, :]),
          fc_vecs[1:2, :], fc_vecs[2:3, :])
  h = _ln(jax.nn.relu(_dot_tb(h, l1_w[...]) + l1_vecs[0:1, :]),
          l1_vecs[1:2, :], l1_vecs[2:3, :])
  h = _ln(jax.nn.relu(_dot_tb(h, l2_w[...]) + l2_vecs[0:1, :]),
          l2_vecs[1:2, :], l2_vecs[2:3, :])
  mlp_term = _dot_t(h, mlp_w[0:1, :])

  logit = mf_term + mlp_term + mlp_w[1, 0]
  blk = logit.shape[0]
  out_ref[...] = jax.nn.sigmoid(logit).reshape(blk // 128, 128)


def _full(shape):
  return pl.BlockSpec(shape, lambda i: (0, 0))


def kernel(params, user_id, product_id):
  p = params
  B = user_id.shape[0]
  D = p["mf_user"].shape[1]

  uid = user_id.astype(jnp.int32)
  pid = product_id.astype(jnp.int32)

  gather = _make_sc_gather(B, 2 * D)
  prod_w = _widen_prod(p["mf_prod"].T, p["mlp_prod"].T)
  p_rows = gather(prod_w, pid)
  user_w = _widen_user(p["mf_user"].T)
  uflag = (uid >= _UHALF)
  uq = jnp.where(uflag, uid - _UHALF, uid)
  u_rows = gather(user_w, uq)
  uflag_f = uflag.astype(jnp.float32)[:, None]

  a = p["attn"]
  f0 = p["final_W"][0, 0]
  f1 = p["final_W"][0, 1]
  mf_w = (f0 * p["mf_out_W"][0])[None, :]
  bias_total = (f0 * p["mf_out_b"][0] + f1 * p["mlp_out_b"][0]
                + p["final_b"][0])
  mlp_w = jnp.stack([f1 * p["mlp_out_W"][0],
                     jnp.full((D,), bias_total, jnp.float32)])

  mf_gb = jnp.stack([p["mf_g"], p["mf_b"]])
  mlp_gb = jnp.stack([p["mlp_g"], p["mlp_b"]])
  bvo = jnp.stack([a["bv"], a["bo"]])
  H0, H1, H2 = p["fc_W"].shape[0], p["l1_W"].shape[0], p["l2_W"].shape[0]
  fc_w = p["fc_W"][:, :D]
  fc_vecs = jnp.stack([p["fc_b"], p["fc_g"], p["fc_beta"]])
  l1_vecs = jnp.stack([p["l1_b"], p["l1_g"], p["l1_beta"]])
  l2_vecs = jnp.stack([p["l2_b"], p["l2_g"], p["l2_beta"]])

  BLK = 4096
  grid = B // BLK
  row_spec = pl.BlockSpec((BLK, 2 * D), lambda i: (i, 0))

  out = pl.pallas_call(
      _tc_body,
      grid=(grid,),
      in_specs=[
          row_spec, row_spec,
          pl.BlockSpec((BLK, 1), lambda i: (i, 0)),
          _full((2, D)), _full((2, D)),
          _full((D, D)), _full((D, D)), _full((2, D)), _full((1, D)),
          _full((H0, D)), _full((3, H0)),
          _full((H1, H0)), _full((3, H1)),
          _full((H2, H1)), _full((3, H2)),
          _full((2, D)),
      ],
      out_specs=pl.BlockSpec((BLK // 128, 128), lambda i: (i, 0)),
      out_shape=jax.ShapeDtypeStruct((B // 128, 128), jnp.float32),
  )(u_rows, p_rows, uflag_f,
    mf_gb, mlp_gb,
    a["Wv"].astype(jnp.bfloat16), a["Wo"].astype(jnp.bfloat16), bvo, mf_w,
    fc_w.astype(jnp.bfloat16), fc_vecs,
    p["l1_W"].astype(jnp.bfloat16), l1_vecs,
    p["l2_W"].astype(jnp.bfloat16), l2_vecs,
    mlp_w)
  return out.reshape(B, 1)


# speedup vs baseline: 1.0911x; 1.0192x over previous
import functools

import jax
import jax.numpy as jnp
from jax import lax
from jax.experimental import pallas as pl
from jax.experimental.pallas import tpu as pltpu
from jax.experimental.pallas import tpu_sc as plsc

_NC = 2
_NS = 16
_NW = _NC * _NS


def _make_sc_gather(B, W):
  assert B % _NW == 0
  bpw = B // _NW
  mesh = plsc.VectorSubcoreMesh(core_axis_name="c", subcore_axis_name="s")

  nch = 4
  ch = bpw // nch

  @functools.partial(
      pl.kernel,
      mesh=mesh,
      out_type=jax.ShapeDtypeStruct((B, W), jnp.float32),
      scratch_types=[
          pltpu.VMEM((bpw,), jnp.int32),
          pltpu.VMEM((ch, W), jnp.float32),
          pltpu.VMEM((ch, W), jnp.float32),
          pltpu.SemaphoreType.DMA,
          pltpu.SemaphoreType.DMA,
      ],
  )
  def gather1(tab_hbm, idx_hbm, out, idx_v, r0, r1, s0, s1):
    wid = lax.axis_index("s") * _NC + lax.axis_index("c")
    base = wid * bpw
    pltpu.sync_copy(idx_hbm.at[pl.ds(base, bpw)], idx_v)
    rows = (r0, r1)
    sems = (s0, s1)
    pend = [None, None]
    for c in range(nch):
      b = c % 2
      pend[b] = pltpu.async_copy(
          tab_hbm.at[idx_v.at[pl.ds(c * ch, ch)]], rows[b], sems[b])
      if c >= 1:
        pb = (c - 1) % 2
        pend[pb].wait()
        pltpu.sync_copy(rows[pb], out.at[pl.ds(base + (c - 1) * ch, ch)])
    lb = (nch - 1) % 2
    pend[lb].wait()
    pltpu.sync_copy(rows[lb], out.at[pl.ds(base + (nch - 1) * ch, ch)])

  return gather1


_WIDEN_C = 2048


def _widen_prod_body(mp_ref, ml_ref, pw_ref):
  cat = jnp.concatenate([mp_ref[...], ml_ref[...]], axis=0)
  i0 = lax.broadcasted_iota(jnp.int32, (128, 128), 0)
  i1 = lax.broadcasted_iota(jnp.int32, (128, 128), 1)
  eye = (i0 == i1).astype(jnp.float32)
  pw_ref[...] = lax.dot_general(cat, eye, (((0,), (0,)), ((), ())),
                                preferred_element_type=jnp.float32)


_UHALF = 51200


def _widen_prod(mpT, mlT):
  D, V = mpT.shape
  C = _WIDEN_C
  return pl.pallas_call(
      _widen_prod_body,
      grid=(pl.cdiv(V, C),),
      in_specs=[pl.BlockSpec((D, C), lambda i: (0, i))] * 2,
      out_specs=pl.BlockSpec((C, 2 * D), lambda i: (i, 0)),
      out_shape=jax.ShapeDtypeStruct((V, 2 * D), jnp.float32),
  )(mpT, mlT)


def _widen_user(uT):
  D, V = uT.shape
  C = _WIDEN_C
  hb = _UHALF // C
  lastb = (V - 1) // C

  def body(a_ref, b_ref, uw_ref):
    i = pl.program_id(0)
    base2 = jnp.minimum(hb + i, lastb) * C
    col = base2 + lax.broadcasted_iota(jnp.int32, (D, C), 1)
    b = jnp.where(col < V, b_ref[...], 0.0)
    cat = jnp.concatenate([a_ref[...], b], axis=0)
    i0 = lax.broadcasted_iota(jnp.int32, (2 * D, 2 * D), 0)
    i1 = lax.broadcasted_iota(jnp.int32, (2 * D, 2 * D), 1)
    eye = (i0 == i1).astype(jnp.float32)
    uw_ref[...] = lax.dot_general(cat, eye, (((0,), (0,)), ((), ())),
                                  preferred_element_type=jnp.float32)

  return pl.pallas_call(
      body,
      grid=(hb,),
      in_specs=[
          pl.BlockSpec((D, C), lambda i: (0, i)),
          pl.BlockSpec((D, C), lambda i: (0, jnp.minimum(hb + i, lastb))),
      ],
      out_specs=pl.BlockSpec((C, 2 * D), lambda i: (i, 0)),
      out_shape=jax.ShapeDtypeStruct((_UHALF, 2 * D), jnp.float32),
  )(uT, uT)


def _ln(x, g, b, eps=1e-5):
  n = x.shape[-1]
  ones = jnp.full((n, 1), 1.0 / n, jnp.float32)
  dn = (((1,), (0,)), ((), ()))
  m = lax.dot_general(x, ones, dn, preferred_element_type=jnp.float32)
  q = lax.dot_general(x * x, ones, dn, preferred_element_type=jnp.float32)
  s = lax.rsqrt(q - m * m + eps)
  sg = lax.dot_general(s, g, dn, preferred_element_type=jnp.float32)
  mg = lax.dot_general(m * s, g, dn, preferred_element_type=jnp.float32)
  return x * sg - mg + b


def _dot_t(x, w):
  return lax.dot_general(x, w, (((1,), (1,)), ((), ())),
                         preferred_element_type=jnp.float32)


def _dot_tb(x, w_bf):
  return lax.dot_general(x.astype(jnp.bfloat16), w_bf,
                         (((1,), (1,)), ((), ())),
                         preferred_element_type=jnp.float32)


def _tc_body(u_rows, p_rows, uflag,
             mf_gb, mlp_gb, wv, wo, bvo, head_w,
             fc_w, fc_vecs, l1_w, l1_vecs, l2_w, l2_vecs,
             out_ref):
  mf_g = mf_gb[0:1, :]
  mf_b = mf_gb[1:2, :]
  mlp_g = mlp_gb[0:1, :]
  mlp_b = mlp_gb[1:2, :]

  uL = u_rows[:, :64]
  uR = u_rows[:, 64:]
  fb = lax.dot_general(uflag[...], jnp.ones((1, 64), jnp.float32),
                       (((1,), (0,)), ((), ())),
                       preferred_element_type=jnp.float32)
  u_mf = uL + fb * (uR - uL)
  p_mf = p_rows[:, :64]
  p_mlp = p_rows[:, 64:]

  mf_vec = _ln(u_mf, mf_g, mf_b) * _ln(p_mf, mf_g, mf_b)
  mf_term = _dot_t(mf_vec, head_w[0:1, :])

  x = _ln(p_mlp, mlp_g, mlp_b)
  a = _dot_tb(x, wv[...]) + bvo[0:1, :]
  a = _dot_tb(a, wo[...]) + bvo[1:2, :]

  h = _ln(jax.nn.relu(_dot_tb(a, fc_w[...]) + fc_vecs[0:1, :]),
          fc_vecs[1:2, :], fc_vecs[2:3, :])
  h = _ln(jax.nn.relu(_dot_tb(h, l1_w[...]) + l1_vecs[0:1, :]),
          l1_vecs[1:2, :], l1_vecs[2:3, :])
  h = _ln(jax.nn.relu(_dot_tb(h, l2_w[...]) + l2_vecs[0:1, :]),
          l2_vecs[1:2, :], l2_vecs[2:3, :])
  mlp_term = _dot_t(h, head_w[1:2, :])

  logit = mf_term + mlp_term + bvo[2, 0]
  blk = logit.shape[0]
  out_ref[...] = jax.nn.sigmoid(logit).reshape(blk // 128, 128)


def _full(shape):
  return pl.BlockSpec(shape, lambda i: (0, 0))


def kernel(params, user_id, product_id):
  p = params
  B = user_id.shape[0]
  D = p["mf_user"].shape[1]

  uid = user_id.astype(jnp.int32)
  pid = product_id.astype(jnp.int32)

  gather = _make_sc_gather(B, 2 * D)
  prod_w = _widen_prod(p["mf_prod"].T, p["mlp_prod"].T)
  p_rows = gather(prod_w, pid)
  user_w = _widen_user(p["mf_user"].T)
  uflag = (uid >= _UHALF)
  uq = jnp.where(uflag, uid - _UHALF, uid)
  u_rows = gather(user_w, uq)
  uflag_f = uflag.astype(jnp.float32)[:, None]

  a = p["attn"]
  f0 = p["final_W"][0, 0]
  f1 = p["final_W"][0, 1]
  bias_total = (f0 * p["mf_out_b"][0] + f1 * p["mlp_out_b"][0]
                + p["final_b"][0])
  head_w = jnp.stack([f0 * p["mf_out_W"][0],
                      f1 * p["mlp_out_W"][0]])

  mf_gb = jnp.stack([p["mf_g"], p["mf_b"]])
  mlp_gb = jnp.stack([p["mlp_g"], p["mlp_b"]])
  bvo = jnp.stack([a["bv"], a["bo"],
                   jnp.full((D,), bias_total, jnp.float32)])
  H0, H1, H2 = p["fc_W"].shape[0], p["l1_W"].shape[0], p["l2_W"].shape[0]
  fc_w = p["fc_W"][:, :D]
  fc_vecs = jnp.stack([p["fc_b"], p["fc_g"], p["fc_beta"]])
  l1_vecs = jnp.stack([p["l1_b"], p["l1_g"], p["l1_beta"]])
  l2_vecs = jnp.stack([p["l2_b"], p["l2_g"], p["l2_beta"]])

  BLK = 4096
  grid = B // BLK
  row_spec = pl.BlockSpec((BLK, 2 * D), lambda i: (i, 0))

  out = pl.pallas_call(
      _tc_body,
      grid=(grid,),
      in_specs=[
          row_spec, row_spec,
          pl.BlockSpec((BLK, 1), lambda i: (i, 0)),
          _full((2, D)), _full((2, D)),
          _full((D, D)), _full((D, D)), _full((3, D)), _full((2, D)),
          _full((H0, D)), _full((3, H0)),
          _full((H1, H0)), _full((3, H1)),
          _full((H2, H1)), _full((3, H2)),
      ],
      out_specs=pl.BlockSpec((BLK // 128, 128), lambda i: (i, 0)),
      out_shape=jax.ShapeDtypeStruct((B // 128, 128), jnp.float32),
  )(u_rows, p_rows, uflag_f,
    mf_gb, mlp_gb,
    a["Wv"].astype(jnp.bfloat16), a["Wo"].astype(jnp.bfloat16), bvo, head_w,
    fc_w.astype(jnp.bfloat16), fc_vecs,
    p["l1_W"].astype(jnp.bfloat16), l1_vecs,
    p["l2_W"].astype(jnp.bfloat16), l2_vecs)
  return out.reshape(B, 1)
